# baseline (speedup 1.0000x reference)
import functools

import jax
import jax.numpy as jnp
from jax import lax
from jax.experimental import pallas as pl
from jax.experimental.pallas import tpu as pltpu
from jax.experimental.pallas import tpu_sc as plsc

_NC = 2
_NS = 16
_NW = _NC * _NS
_STAGERS = 10
_STAGE_ROUND = 10_000


def _sc_gather(idx_flat, values):
    total = idx_flat.shape[0]
    nvals = values.shape[0]
    assert total % (8 * _NW) == 0
    per_w = total // _NW
    stage_per = nvals // _STAGERS
    assert stage_per % _STAGE_ROUND == 0 and _STAGE_ROUND % 8 == 0
    nrounds = stage_per // _STAGE_ROUND
    mesh = plsc.VectorSubcoreMesh(core_axis_name="c", subcore_axis_name="s")

    @functools.partial(
        pl.kernel,
        mesh=mesh,
        out_type=jax.ShapeDtypeStruct((total,), jnp.float32),
        scratch_types=[
            pltpu.VMEM_SHARED((nvals,), jnp.float32),
            pltpu.VMEM((per_w,), jnp.int32),
            pltpu.VMEM((per_w,), jnp.float32),
            pltpu.SemaphoreType.DMA,
            pltpu.SemaphoreType.DMA,
            pltpu.SemaphoreType.DMA,
        ],
    )
    def k(idx_hbm, values_hbm, out_hbm, shared, idx_v, rows_v, sem, isem,
          stsem):
        c = lax.axis_index("c")
        s = lax.axis_index("s")
        wid = s * _NC + c
        base = wid * per_w
        idx_cp = pltpu.async_copy(idx_hbm.at[pl.ds(base, per_w)], idx_v, isem)

        @pl.when(s < _STAGERS)
        def _stage():
            r = _STAGE_ROUND
            bufs = [rows_v.at[pl.ds(0, r)], rows_v.at[pl.ds(r, r)]]
            my0 = s * stage_per
            ins = [None] * nrounds
            outs = [None] * nrounds
            ins[0] = pltpu.async_copy(values_hbm.at[pl.ds(my0, r)],
                                      bufs[0], stsem)
            for j in range(nrounds):
                ins[j].wait()
                outs[j] = pltpu.async_copy(
                    bufs[j % 2], shared.at[pl.ds(my0 + j * r, r)], sem)
                if j + 1 < nrounds:
                    if j >= 1:
                        outs[j - 1].wait()
                    ins[j + 1] = pltpu.async_copy(
                        values_hbm.at[pl.ds(my0 + (j + 1) * r, r)],
                        bufs[(j + 1) % 2], stsem)
            outs[nrounds - 2].wait()
            outs[nrounds - 1].wait()

        plsc.subcore_barrier()
        idx_cp.wait()
        pltpu.async_copy(shared.at[idx_v], rows_v, sem).wait()
        pltpu.sync_copy(rows_v, out_hbm.at[pl.ds(base, per_w)])

    return k(idx_flat, values)


def kernel(indices, values):
    b, l = indices.shape
    idx_flat = indices.T.reshape(-1).astype(jnp.int32)
    out = _sc_gather(idx_flat, values)
    return out.reshape(l, b, 1).transpose(1, 0, 2)
